# baseline (speedup 1.0000x reference)
import functools

import jax
import jax.numpy as jnp
from jax import lax
from jax.experimental import pallas as pl
from jax.experimental.pallas import tpu as pltpu
from jax.experimental.pallas import tpu_sc as plsc

BATCH = 4096
EMBED = 32
NC, NS = 2, 16
NW = NC * NS
B_PER_W = BATCH // NW


def _make_sc_gather():
    mesh = plsc.VectorSubcoreMesh(core_axis_name="c", subcore_axis_name="s")

    @functools.partial(
        pl.kernel,
        mesh=mesh,
        out_type=(
            jax.ShapeDtypeStruct((NW, EMBED, B_PER_W), jnp.float32),
            jax.ShapeDtypeStruct((NW, EMBED, B_PER_W), jnp.float32),
        ),
        scratch_types=[
            pltpu.VMEM((B_PER_W,), jnp.int32),
            pltpu.VMEM((B_PER_W,), jnp.int32),
            pltpu.VMEM((EMBED, B_PER_W), jnp.int32),
            pltpu.VMEM((EMBED, B_PER_W), jnp.int32),
            pltpu.VMEM((EMBED, B_PER_W), jnp.float32),
            pltpu.VMEM((EMBED, B_PER_W), jnp.float32),
            pltpu.SemaphoreType.DMA,
        ],
    )
    def gather2(uid_hbm, pid_hbm, utab_hbm, ptab_hbm, uout_hbm, pout_hbm,
                uidx_v, pidx_v, uix2_v, pix2_v, uvalsT_v, pvalsT_v, sem):
        wid = lax.axis_index("s") * NC + lax.axis_index("c")
        base = wid * B_PER_W
        pltpu.sync_copy(uid_hbm.at[pl.ds(base, B_PER_W)], uidx_v)
        pltpu.sync_copy(pid_hbm.at[pl.ds(base, B_PER_W)], pidx_v)

        @pl.loop(0, B_PER_W, step=16)
        def _(j):
            su = uidx_v[pl.ds(j, 16)] * EMBED
            sp = pidx_v[pl.ds(j, 16)] * EMBED
            for c in range(EMBED):
                uix2_v[c, pl.ds(j, 16)] = su + c
                pix2_v[c, pl.ds(j, 16)] = sp + c

        copies = []
        for c in range(EMBED):
            copies.append(
                pltpu.async_copy(utab_hbm.at[uix2_v.at[c]], uvalsT_v.at[c], sem))
            copies.append(
                pltpu.async_copy(ptab_hbm.at[pix2_v.at[c]], pvalsT_v.at[c], sem))
        for cp in copies:
            cp.wait()

        pltpu.sync_copy(uvalsT_v, uout_hbm.at[wid])
        pltpu.sync_copy(pvalsT_v, pout_hbm.at[wid])

    return gather2


_sc_gather2 = _make_sc_gather()


def _mlp_body(uT_ref, pT_ref, w1u_ref, w1p_ref, b1_ref, w2_ref, b2_ref,
              w3_ref, b3_ref, out_ref):
    dot = functools.partial(jnp.dot, preferred_element_type=jnp.float32,
                            precision=lax.Precision.HIGHEST)
    dotg = functools.partial(lax.dot_general,
                             dimension_numbers=(((0,), (0,)), ((), ())),
                             preferred_element_type=jnp.float32,
                             precision=lax.Precision.HIGHEST)
    for w in range(NW):
        xu = uT_ref[w]
        xp = pT_ref[w]
        h1 = dotg(xu, w1u_ref[...]) + dotg(xp, w1p_ref[...])
        h1 = jnp.maximum(h1 + b1_ref[...], 0.0)
        h2 = jnp.maximum(dot(h1, w2_ref[...]) + b2_ref[...], 0.0)
        out_ref[pl.ds(w * B_PER_W, B_PER_W), :] = (
            dot(h2, w3_ref[...]) + b3_ref[...])


def _mlp(uT3, pT3, W1, b1, W2, b2, W3, b3):
    return pl.pallas_call(
        _mlp_body,
        out_shape=jax.ShapeDtypeStruct((BATCH, 1), jnp.float32),
    )(uT3, pT3, W1[:EMBED], W1[EMBED:], b1.reshape(1, -1),
      W2, b2.reshape(1, -1), W3, b3.reshape(1, -1))


def kernel(user_ids, product_ids, user_table, prod_table, W1, b1, W2, b2, W3, b3):
    uT3, pT3 = _sc_gather2(user_ids.astype(jnp.int32),
                           product_ids.astype(jnp.int32),
                           user_table.reshape(-1), prod_table.reshape(-1))
    return _mlp(uT3, pT3, W1, b1, W2, b2, W3, b3)


# speedup vs baseline: 1.4962x; 1.4962x over previous
import dataclasses
import functools

import jax
import jax.numpy as jnp
from jax import lax
from jax.experimental import pallas as pl
from jax.experimental.pallas import tpu as pltpu
from jax.experimental.pallas import tpu_sc as plsc

BATCH = 4096
EMBED = 32
NC, NS = 2, 16
NW = NC * NS
B_PER_W = BATCH // NW
R_PER_ROUND = 16
N_ROUNDS = B_PER_W // R_PER_ROUND


def _make_sc_gather():
    mesh = plsc.VectorSubcoreMesh(core_axis_name="c", subcore_axis_name="s")

    cp = pltpu.CompilerParams()
    if "needs_layout_passes" in pltpu.CompilerParams.__dataclass_fields__:
        cp = dataclasses.replace(cp, needs_layout_passes=False)

    blk = (R_PER_ROUND * 8, EMBED)

    @functools.partial(
        pl.kernel,
        mesh=mesh,
        compiler_params=cp,
        out_type=(
            jax.ShapeDtypeStruct((BATCH, EMBED), jnp.float32),
            jax.ShapeDtypeStruct((BATCH, EMBED), jnp.float32),
        ),
        scratch_types=[
            pltpu.VMEM((B_PER_W,), jnp.int32),
            pltpu.VMEM((B_PER_W,), jnp.int32),
            pltpu.VMEM(blk, jnp.float32),
            pltpu.VMEM(blk, jnp.float32),
            pltpu.VMEM(blk, jnp.float32),
            pltpu.VMEM(blk, jnp.float32),
            pltpu.VMEM((B_PER_W, EMBED), jnp.float32),
            pltpu.VMEM((B_PER_W, EMBED), jnp.float32),
            pltpu.SemaphoreType.DMA,
        ],
    )
    def gather2(uid_hbm, pid_hbm, utab_hbm, ptab_hbm, uout_hbm, pout_hbm,
                uidx_v, pidx_v, ubuf0, ubuf1, pbuf0, pbuf1,
                urows_v, prows_v, sem):
        wid = lax.axis_index("s") * NC + lax.axis_index("c")
        base = wid * B_PER_W
        pltpu.sync_copy(uid_hbm.at[pl.ds(base, B_PER_W)], uidx_v)
        pltpu.sync_copy(pid_hbm.at[pl.ds(base, B_PER_W)], pidx_v)

        ubufs = (ubuf0, ubuf1)
        pbufs = (pbuf0, pbuf1)
        iota16 = lax.iota(jnp.int32, 16)

        def issue(r):
            ub, pb = ubufs[r % 2], pbufs[r % 2]
            uv = (uidx_v[pl.ds(r * R_PER_ROUND, 16)] >> 3) * 8
            pv = (pidx_v[pl.ds(r * R_PER_ROUND, 16)] >> 3) * 8
            for t in range(R_PER_ROUND):
                ut = pl.multiple_of(uv[t], 8)
                pt = pl.multiple_of(pv[t], 8)
                pltpu.async_copy(utab_hbm.at[pl.ds(ut, 8)],
                                 ub.at[pl.ds(t * 8, 8)], sem)
                pltpu.async_copy(ptab_hbm.at[pl.ds(pt, 8)],
                                 pb.at[pl.ds(t * 8, 8)], sem)

        def drain_and_select(r):
            ub, pb = ubufs[r % 2], pbufs[r % 2]
            for t in range(R_PER_ROUND):
                pltpu.make_async_copy(utab_hbm.at[pl.ds(0, 8)],
                                      ub.at[pl.ds(t * 8, 8)], sem).wait()
                pltpu.make_async_copy(ptab_hbm.at[pl.ds(0, 8)],
                                      pb.at[pl.ds(t * 8, 8)], sem).wait()
            usub = uidx_v[pl.ds(r * R_PER_ROUND, 16)] & 7
            psub = pidx_v[pl.ds(r * R_PER_ROUND, 16)] & 7
            urvec = iota16 * 8 + usub
            prvec = iota16 * 8 + psub
            orow = r * R_PER_ROUND + iota16
            for c in range(EMBED):
                cvec = jnp.full((16,), c, jnp.int32)
                uvals = plsc.load_gather(ub, [urvec, cvec])
                pvals = plsc.load_gather(pb, [prvec, cvec])
                plsc.store_scatter(urows_v, [orow, cvec], uvals)
                plsc.store_scatter(prows_v, [orow, cvec], pvals)

        issue(0)
        for r in range(1, N_ROUNDS):
            issue(r)
            drain_and_select(r - 1)
        drain_and_select(N_ROUNDS - 1)

        pltpu.sync_copy(urows_v, uout_hbm.at[pl.ds(base, B_PER_W)])
        pltpu.sync_copy(prows_v, pout_hbm.at[pl.ds(base, B_PER_W)])

    return gather2


_sc_gather2 = _make_sc_gather()


def _mlp_body(u_ref, p_ref, w1u_ref, w1p_ref, b1_ref, w2_ref, b2_ref,
              w3_ref, b3_ref, out_ref):
    dot = functools.partial(jnp.dot, preferred_element_type=jnp.float32)
    h1 = dot(u_ref[...], w1u_ref[...]) + dot(p_ref[...], w1p_ref[...])
    h1 = jnp.maximum(h1 + b1_ref[...], 0.0)
    h2 = jnp.maximum(dot(h1, w2_ref[...]) + b2_ref[...], 0.0)
    out_ref[...] = dot(h2, w3_ref[...]) + b3_ref[...]


def _mlp(u_emb, p_emb, W1, b1, W2, b2, W3, b3):
    return pl.pallas_call(
        _mlp_body,
        out_shape=jax.ShapeDtypeStruct((BATCH, 1), jnp.float32),
    )(u_emb, p_emb, W1[:EMBED], W1[EMBED:], b1.reshape(1, -1),
      W2, b2.reshape(1, -1), W3, b3.reshape(1, -1))


def kernel(user_ids, product_ids, user_table, prod_table, W1, b1, W2, b2, W3, b3):
    u_emb, p_emb = _sc_gather2(user_ids.astype(jnp.int32),
                               product_ids.astype(jnp.int32),
                               user_table, prod_table)
    return _mlp(u_emb, p_emb, W1, b1, W2, b2, W3, b3)


# speedup vs baseline: 1.5746x; 1.0524x over previous
import dataclasses
import functools

import jax
import jax.numpy as jnp
from jax import lax
from jax.experimental import pallas as pl
from jax.experimental.pallas import tpu as pltpu
from jax.experimental.pallas import tpu_sc as plsc

BATCH = 4096
EMBED = 32
NC, NS = 2, 16
NW = NC * NS
B_PER_W = BATCH // NW
R_PER_ROUND = 16
N_ROUNDS = B_PER_W // R_PER_ROUND


def _make_sc_gather():
    mesh = plsc.VectorSubcoreMesh(core_axis_name="c", subcore_axis_name="s")

    cp = pltpu.CompilerParams()
    if "needs_layout_passes" in pltpu.CompilerParams.__dataclass_fields__:
        cp = dataclasses.replace(cp, needs_layout_passes=False)

    blk = (R_PER_ROUND * 8, EMBED)

    @functools.partial(
        pl.kernel,
        mesh=mesh,
        compiler_params=cp,
        out_type=jax.ShapeDtypeStruct((BATCH, EMBED), jnp.float32),
        scratch_types=[
            pltpu.VMEM((B_PER_W,), jnp.int32),
            pltpu.VMEM(blk, jnp.float32),
            pltpu.VMEM(blk, jnp.float32),
            pltpu.VMEM((B_PER_W, EMBED), jnp.float32),
            pltpu.SemaphoreType.DMA,
        ],
    )
    def gather1(ids_hbm, tab_hbm, out_hbm, idx_v, buf0, buf1, rows_v, sem):
        wid = lax.axis_index("s") * NC + lax.axis_index("c")
        base = wid * B_PER_W
        pltpu.sync_copy(ids_hbm.at[pl.ds(base, B_PER_W)], idx_v)

        bufs = (buf0, buf1)
        iota16 = lax.iota(jnp.int32, 16)

        def issue(r):
            b = bufs[r % 2]
            v = (idx_v[pl.ds(r * R_PER_ROUND, 16)] >> 3) * 8
            for t in range(R_PER_ROUND):
                tb = pl.multiple_of(v[t], 8)
                pltpu.async_copy(tab_hbm.at[pl.ds(tb, 8)],
                                 b.at[pl.ds(t * 8, 8)], sem)

        def drain_and_select(r):
            b = bufs[r % 2]
            pltpu.make_async_copy(tab_hbm.at[pl.ds(0, R_PER_ROUND * 8)],
                                  b, sem).wait()
            sub = idx_v[pl.ds(r * R_PER_ROUND, 16)] & 7
            rvec = iota16 * 8 + sub
            orow = r * R_PER_ROUND + iota16
            for c in range(EMBED):
                cvec = jnp.full((16,), c, jnp.int32)
                vals = plsc.load_gather(b, [rvec, cvec])
                plsc.store_scatter(rows_v, [orow, cvec], vals)

        issue(0)
        for r in range(1, N_ROUNDS):
            issue(r)
            drain_and_select(r - 1)
        drain_and_select(N_ROUNDS - 1)

        pltpu.sync_copy(rows_v, out_hbm.at[pl.ds(base, B_PER_W)])

    return gather1


_sc_gather1 = _make_sc_gather()


def _mlp_body(u_ref, p_ref, w1u_ref, w1p_ref, b1_ref, w2_ref, b2_ref,
              w3_ref, b3_ref, out_ref):
    dot = functools.partial(jnp.dot, preferred_element_type=jnp.float32)
    h1 = dot(u_ref[...], w1u_ref[...]) + dot(p_ref[...], w1p_ref[...])
    h1 = jnp.maximum(h1 + b1_ref[...], 0.0)
    h2 = jnp.maximum(dot(h1, w2_ref[...]) + b2_ref[...], 0.0)
    out_ref[...] = dot(h2, w3_ref[...]) + b3_ref[...]


def _mlp(u_emb, p_emb, W1, b1, W2, b2, W3, b3):
    return pl.pallas_call(
        _mlp_body,
        out_shape=jax.ShapeDtypeStruct((BATCH, 1), jnp.float32),
    )(u_emb, p_emb, W1[:EMBED], W1[EMBED:], b1.reshape(1, -1),
      W2, b2.reshape(1, -1), W3, b3.reshape(1, -1))


def kernel(user_ids, product_ids, user_table, prod_table, W1, b1, W2, b2, W3, b3):
    u_emb = _sc_gather1(user_ids.astype(jnp.int32), user_table)
    p_emb = _sc_gather1(product_ids.astype(jnp.int32), prod_table)
    return _mlp(u_emb, p_emb, W1, b1, W2, b2, W3, b3)


# speedup vs baseline: 1.5751x; 1.0003x over previous
import dataclasses
import functools

import jax
import jax.numpy as jnp
from jax import lax
from jax.experimental import pallas as pl
from jax.experimental.pallas import tpu as pltpu
from jax.experimental.pallas import tpu_sc as plsc

BATCH = 4096
EMBED = 32
NC, NS = 2, 16
NW = NC * NS
B_PER_W = BATCH // NW
R_PER_ROUND = 16
N_ROUNDS = B_PER_W // R_PER_ROUND


def _make_sc_gather():
    mesh = plsc.VectorSubcoreMesh(core_axis_name="c", subcore_axis_name="s")

    cp = pltpu.CompilerParams()
    if "needs_layout_passes" in pltpu.CompilerParams.__dataclass_fields__:
        cp = dataclasses.replace(cp, needs_layout_passes=False)

    blk = (R_PER_ROUND * 8, EMBED)

    @functools.partial(
        pl.kernel,
        mesh=mesh,
        compiler_params=cp,
        out_type=jax.ShapeDtypeStruct((BATCH, EMBED), jnp.float32),
        scratch_types=[
            pltpu.VMEM((B_PER_W,), jnp.int32),
            pltpu.VMEM(blk, jnp.float32),
            pltpu.VMEM(blk, jnp.float32),
            pltpu.VMEM((B_PER_W, EMBED), jnp.float32),
            pltpu.SemaphoreType.DMA,
        ],
    )
    def gather1(ids_hbm, tab_hbm, out_hbm, idx_v, buf0, buf1, rows_v, sem):
        wid = lax.axis_index("s") * NC + lax.axis_index("c")
        base = wid * B_PER_W
        pltpu.sync_copy(ids_hbm.at[pl.ds(base, B_PER_W)], idx_v)

        bufs = (buf0, buf1)
        iota16 = lax.iota(jnp.int32, 16)

        def issue(r):
            b = bufs[r % 2]
            v = (idx_v[pl.ds(r * R_PER_ROUND, 16)] >> 3) * 8
            for t in range(R_PER_ROUND):
                tb = pl.multiple_of(v[t], 8)
                pltpu.async_copy(tab_hbm.at[pl.ds(tb, 8)],
                                 b.at[pl.ds(t * 8, 8)], sem)

        def drain_and_select(r):
            b = bufs[r % 2]
            pltpu.make_async_copy(tab_hbm.at[pl.ds(0, R_PER_ROUND * 8)],
                                  b, sem).wait()
            sub = idx_v[pl.ds(r * R_PER_ROUND, 16)] & 7
            rvec = iota16 * 8 + sub
            orow = r * R_PER_ROUND + iota16
            for c in range(EMBED):
                cvec = jnp.full((16,), c, jnp.int32)
                vals = plsc.load_gather(b, [rvec, cvec])
                plsc.store_scatter(rows_v, [orow, cvec], vals)

        issue(0)
        for r in range(1, N_ROUNDS):
            issue(r)
            drain_and_select(r - 1)
        drain_and_select(N_ROUNDS - 1)

        pltpu.sync_copy(rows_v, out_hbm.at[pl.ds(base, B_PER_W)])

    return gather1


_sc_gather1 = _make_sc_gather()


def _mlp_body(u_ref, p_ref, w1u_ref, w1p_ref, b1_ref, w2_ref, b2_ref,
              w3_ref, b3_ref, out_ref):
    dot = functools.partial(jnp.dot, preferred_element_type=jnp.float32)
    bf = jnp.bfloat16
    h1 = (dot(u_ref[...].astype(bf), w1u_ref[...].astype(bf))
          + dot(p_ref[...].astype(bf), w1p_ref[...].astype(bf)))
    h1 = jnp.maximum(h1 + b1_ref[...], 0.0)
    h2 = jnp.maximum(dot(h1.astype(bf), w2_ref[...].astype(bf)) + b2_ref[...],
                     0.0)
    out_ref[...] = dot(h2, w3_ref[...]) + b3_ref[...]


def _mlp(u_emb, p_emb, W1, b1, W2, b2, W3, b3):
    return pl.pallas_call(
        _mlp_body,
        out_shape=jax.ShapeDtypeStruct((BATCH, 1), jnp.float32),
    )(u_emb, p_emb, W1[:EMBED], W1[EMBED:], b1.reshape(1, -1),
      W2, b2.reshape(1, -1), W3, b3.reshape(1, -1))


def kernel(user_ids, product_ids, user_table, prod_table, W1, b1, W2, b2, W3, b3):
    u_emb = _sc_gather1(user_ids.astype(jnp.int32), user_table)
    p_emb = _sc_gather1(product_ids.astype(jnp.int32), prod_table)
    return _mlp(u_emb, p_emb, W1, b1, W2, b2, W3, b3)


# speedup vs baseline: 1.5770x; 1.0012x over previous
import dataclasses
import functools

import jax
import jax.numpy as jnp
from jax import lax
from jax.experimental import pallas as pl
from jax.experimental.pallas import tpu as pltpu
from jax.experimental.pallas import tpu_sc as plsc

BATCH = 4096
EMBED = 32
NC, NS = 2, 16
NW = NC * NS
B_PER_W = BATCH // NW
R_PER_ROUND = 16
N_ROUNDS = B_PER_W // R_PER_ROUND


def _make_sc_gather():
    mesh = plsc.VectorSubcoreMesh(core_axis_name="c", subcore_axis_name="s")

    cp = pltpu.CompilerParams()
    if "needs_layout_passes" in pltpu.CompilerParams.__dataclass_fields__:
        cp = dataclasses.replace(cp, needs_layout_passes=False)

    blk = (R_PER_ROUND * 8, EMBED)

    @functools.partial(
        pl.kernel,
        mesh=mesh,
        compiler_params=cp,
        out_type=jax.ShapeDtypeStruct((BATCH, EMBED), jnp.float32),
        scratch_types=[
            pltpu.VMEM((B_PER_W,), jnp.int32),
            pltpu.VMEM(blk, jnp.float32),
            pltpu.VMEM(blk, jnp.float32),
            pltpu.VMEM((B_PER_W, EMBED), jnp.float32),
            pltpu.SemaphoreType.DMA,
        ],
    )
    def gather1(ids_hbm, tab_hbm, out_hbm, idx_v, buf0, buf1, rows_v, sem):
        wid = lax.axis_index("s") * NC + lax.axis_index("c")
        base = wid * B_PER_W
        pltpu.sync_copy(ids_hbm.at[pl.ds(base, B_PER_W)], idx_v)

        bufs = (buf0, buf1)
        iota16 = lax.iota(jnp.int32, 16)

        def issue(r):
            b = bufs[r % 2]
            v = (idx_v[pl.ds(r * R_PER_ROUND, 16)] >> 3) * 8
            for t in range(R_PER_ROUND):
                tb = pl.multiple_of(v[t], 8)
                pltpu.async_copy(tab_hbm.at[pl.ds(tb, 8)],
                                 b.at[pl.ds(t * 8, 8)], sem)

        def drain_and_select(r):
            b = bufs[r % 2]
            pltpu.make_async_copy(tab_hbm.at[pl.ds(0, R_PER_ROUND * 8)],
                                  b, sem).wait()
            sub = idx_v[pl.ds(r * R_PER_ROUND, 16)] & 7
            rvec = iota16 * 8 + sub
            orow = r * R_PER_ROUND + iota16
            for c in range(EMBED):
                cvec = jnp.full((16,), c, jnp.int32)
                vals = plsc.load_gather(b, [rvec, cvec])
                plsc.store_scatter(rows_v, [orow, cvec], vals)

        issue(0)
        for r in range(1, N_ROUNDS):
            issue(r)
            drain_and_select(r - 1)
        drain_and_select(N_ROUNDS - 1)

        pltpu.sync_copy(rows_v, out_hbm.at[pl.ds(base, B_PER_W)])

    return gather1


_sc_gather1 = _make_sc_gather()


def _mlp_body(u_ref, p_ref, w1u_ref, w1p_ref, b1_ref, w2_ref, b2_ref,
              w3_ref, b3_ref, out_ref):
    dot = functools.partial(jnp.dot, preferred_element_type=jnp.float32)
    h1 = dot(u_ref[...], w1u_ref[...]) + dot(p_ref[...], w1p_ref[...])
    h1 = jnp.maximum(h1 + b1_ref[...], 0.0)
    h2 = jnp.maximum(dot(h1, w2_ref[...]) + b2_ref[...], 0.0)
    out_ref[...] = dot(h2, w3_ref[...]) + b3_ref[...]


def _mlp(u_emb, p_emb, W1, b1, W2, b2, W3, b3):
    return pl.pallas_call(
        _mlp_body,
        out_shape=jax.ShapeDtypeStruct((BATCH, 1), jnp.float32),
    )(u_emb, p_emb, W1[:EMBED], W1[EMBED:], b1.reshape(1, -1),
      W2, b2.reshape(1, -1), W3, b3.reshape(1, -1))


def kernel(user_ids, product_ids, user_table, prod_table, W1, b1, W2, b2, W3, b3):
    u_emb = _sc_gather1(user_ids.astype(jnp.int32), user_table)
    p_emb = _sc_gather1(product_ids.astype(jnp.int32), prod_table)
    return _mlp(u_emb, p_emb, W1, b1, W2, b2, W3, b3)
